# baseline (device time: 23404 ns/iter reference)
import jax
import jax.numpy as jnp
from jax import lax
from jax.experimental import pallas as pl
from jax.experimental.pallas import tpu as pltpu

N_DEV = 4
B, SQ, SKV = 2, 128, 128
HQ_LOCAL, DH = 4, 64
D_MODEL = 512
CHUNK = HQ_LOCAL * DH
ROWS = B * SQ


def kernel(x, Wq, K_ext, V_ext, Wo):
    my_pos = lax.axis_index("i")
    Wq_loc = lax.dynamic_slice_in_dim(Wq, my_pos * CHUNK, CHUNK, axis=1)

    def body(x_ref, wq_ref, k_ref, v_ref, wo_ref, out_ref,
             comm_ref, send_sems, recv_sems):
        me = lax.axis_index("i")
        left = (me - 1) % N_DEV
        right = (me + 1) % N_DEV

        barrier_sem = pltpu.get_barrier_semaphore()
        for nbr in (left, right):
            pl.semaphore_signal(
                barrier_sem, inc=1,
                device_id=(nbr,), device_id_type=pl.DeviceIdType.MESH,
            )
        pl.semaphore_wait(barrier_sem, 2)

        x2d = x_ref[...].reshape(ROWS, D_MODEL)
        q_all = jnp.dot(x2d, wq_ref[...],
                        preferred_element_type=jnp.float32)

        ii = lax.broadcasted_iota(jnp.int32, (SQ, SKV), 0)
        jj = lax.broadcasted_iota(jnp.int32, (SQ, SKV), 1)
        mask = (jj // 64) <= (ii // 64)

        for b in range(B):
            k2d = k_ref[b].reshape(SKV, CHUNK)
            v2d = v_ref[b].reshape(SKV, CHUNK)
            for h in range(HQ_LOCAL):
                qh = q_all[b * SQ:(b + 1) * SQ, h * DH:(h + 1) * DH]
                kh = k2d[:, h * DH:(h + 1) * DH]
                s = lax.dot_general(
                    qh, kh, (((1,), (1,)), ((), ())),
                    preferred_element_type=jnp.float32,
                ) * 0.125
                s = jnp.where(mask, s, -1e9)
                m = jnp.max(s, axis=1, keepdims=True)
                w = jnp.exp(s - m)
                w = w / jnp.sum(w, axis=1, keepdims=True)
                ctx_h = jnp.dot(w, v2d[:, h * DH:(h + 1) * DH],
                                preferred_element_type=jnp.float32)
                comm_ref[0, b * SQ:(b + 1) * SQ, h * DH:(h + 1) * DH] = ctx_h

        wo_mine = wo_ref[pl.ds(me * CHUNK, CHUNK), :]
        out2d = jnp.dot(comm_ref[0], wo_mine,
                        preferred_element_type=jnp.float32)

        for h in range(N_DEV - 1):
            rdma = pltpu.make_async_remote_copy(
                src_ref=comm_ref.at[h],
                dst_ref=comm_ref.at[h + 1],
                send_sem=send_sems.at[h],
                recv_sem=recv_sems.at[h],
                device_id=(right,),
                device_id_type=pl.DeviceIdType.MESH,
            )
            rdma.start()
            rdma.wait()
            origin = (me - h - 1) % N_DEV
            wo_blk = wo_ref[pl.ds(origin * CHUNK, CHUNK), :]
            out2d = out2d + jnp.dot(comm_ref[h + 1], wo_blk,
                                    preferred_element_type=jnp.float32)

        out_ref[...] = out2d.reshape(B, SQ, D_MODEL)

    return pl.pallas_call(
        body,
        out_shape=jax.ShapeDtypeStruct((B, SQ, D_MODEL), jnp.float32),
        in_specs=[pl.BlockSpec(memory_space=pltpu.VMEM)] * 5,
        out_specs=pl.BlockSpec(memory_space=pltpu.VMEM),
        scratch_shapes=[
            pltpu.VMEM((N_DEV, ROWS, CHUNK), jnp.float32),
            pltpu.SemaphoreType.DMA((N_DEV - 1,)),
            pltpu.SemaphoreType.DMA((N_DEV - 1,)),
        ],
        compiler_params=pltpu.CompilerParams(collective_id=0),
    )(x, Wq_loc, K_ext, V_ext, Wo)


# device time: 17458 ns/iter; 1.3406x vs baseline; 1.3406x over previous
import jax
import jax.numpy as jnp
from jax import lax
from jax.experimental import pallas as pl
from jax.experimental.pallas import tpu as pltpu

N_DEV = 4
B, SQ, SKV = 2, 128, 128
HQ_LOCAL, DH = 4, 64
D_MODEL = 512
CHUNK = HQ_LOCAL * DH
ROWS = B * SQ


def kernel(x, Wq, K_ext, V_ext, Wo):
    my_pos = lax.axis_index("i")
    Wq_loc = lax.dynamic_slice_in_dim(Wq, my_pos * CHUNK, CHUNK, axis=1)

    def body(x_ref, wq_ref, k_ref, v_ref, wo_ref, out_ref,
             comm_ref, send_sems, recv_sems):
        me = lax.axis_index("i")

        barrier_sem = pltpu.get_barrier_semaphore()
        for d in range(1, N_DEV):
            pl.semaphore_signal(
                barrier_sem, inc=1,
                device_id=((me + d) % N_DEV,),
                device_id_type=pl.DeviceIdType.MESH,
            )
        pl.semaphore_wait(barrier_sem, N_DEV - 1)

        x2d = x_ref[...].reshape(ROWS, D_MODEL)
        q_all = jnp.dot(x2d, wq_ref[...],
                        preferred_element_type=jnp.float32)

        ii = lax.broadcasted_iota(jnp.int32, (SQ, SKV), 0)
        jj = lax.broadcasted_iota(jnp.int32, (SQ, SKV), 1)
        mask = (jj // 64) <= (ii // 64)

        for b in range(B):
            k2d = k_ref[b].reshape(SKV, CHUNK)
            v2d = v_ref[b].reshape(SKV, CHUNK)
            for h in range(HQ_LOCAL):
                qh = q_all[b * SQ:(b + 1) * SQ, h * DH:(h + 1) * DH]
                kh = k2d[:, h * DH:(h + 1) * DH]
                s = lax.dot_general(
                    qh, kh, (((1,), (1,)), ((), ())),
                    preferred_element_type=jnp.float32,
                ) * 0.125
                s = jnp.where(mask, s, -1e9)
                m = jnp.max(s, axis=1, keepdims=True)
                w = jnp.exp(s - m)
                w = w / jnp.sum(w, axis=1, keepdims=True)
                ctx_h = jnp.dot(w, v2d[:, h * DH:(h + 1) * DH],
                                preferred_element_type=jnp.float32)
                comm_ref[0, b * SQ:(b + 1) * SQ, h * DH:(h + 1) * DH] = ctx_h

        rdmas = {}
        for d in range(1, N_DEV):
            rdmas[d] = pltpu.make_async_remote_copy(
                src_ref=comm_ref.at[0],
                dst_ref=comm_ref.at[d],
                send_sem=send_sems.at[d - 1],
                recv_sem=recv_sems.at[d - 1],
                device_id=((me + d) % N_DEV,),
                device_id_type=pl.DeviceIdType.MESH,
            )
            rdmas[d].start()

        wo_mine = wo_ref[pl.ds(me * CHUNK, CHUNK), :]
        out2d = jnp.dot(comm_ref[0], wo_mine,
                        preferred_element_type=jnp.float32)

        for d in (1, 3, 2):
            rdmas[d].wait_recv()
            origin = (me - d) % N_DEV
            wo_blk = wo_ref[pl.ds(origin * CHUNK, CHUNK), :]
            out2d = out2d + jnp.dot(comm_ref[d], wo_blk,
                                    preferred_element_type=jnp.float32)
        for d in range(1, N_DEV):
            rdmas[d].wait_send()

        out_ref[...] = out2d.reshape(B, SQ, D_MODEL)

    return pl.pallas_call(
        body,
        out_shape=jax.ShapeDtypeStruct((B, SQ, D_MODEL), jnp.float32),
        in_specs=[pl.BlockSpec(memory_space=pltpu.VMEM)] * 5,
        out_specs=pl.BlockSpec(memory_space=pltpu.VMEM),
        scratch_shapes=[
            pltpu.VMEM((N_DEV, ROWS, CHUNK), jnp.float32),
            pltpu.SemaphoreType.DMA((N_DEV - 1,)),
            pltpu.SemaphoreType.DMA((N_DEV - 1,)),
        ],
        compiler_params=pltpu.CompilerParams(collective_id=0),
    )(x, Wq_loc, K_ext, V_ext, Wo)


# device time: 14716 ns/iter; 1.5904x vs baseline; 1.1863x over previous
import jax
import jax.numpy as jnp
from jax import lax
from jax.experimental import pallas as pl
from jax.experimental.pallas import tpu as pltpu

N_DEV = 4
B, SQ, SKV = 2, 128, 128
HQ_LOCAL, DH = 4, 64
D_MODEL = 512
CHUNK = HQ_LOCAL * DH
ROWS = B * SQ


def kernel(x, Wq, K_ext, V_ext, Wo):
    my_pos = lax.axis_index("i")
    Wq_loc = lax.dynamic_slice_in_dim(Wq, my_pos * CHUNK, CHUNK, axis=1)

    def body(x_ref, wq_ref, k_ref, v_ref, wo_ref, out_ref,
             comm_ref, send_sems, recv_sems):
        me = lax.axis_index("i")

        barrier_sem = pltpu.get_barrier_semaphore()
        for d in range(1, N_DEV):
            pl.semaphore_signal(
                barrier_sem, inc=1,
                device_id=((me + d) % N_DEV,),
                device_id_type=pl.DeviceIdType.MESH,
            )
        pl.semaphore_wait(barrier_sem, N_DEV - 1)

        x2d = x_ref[...].reshape(ROWS, D_MODEL).astype(jnp.bfloat16)
        wq_bf = wq_ref[...].astype(jnp.bfloat16)
        q_all = jnp.dot(x2d, wq_bf,
                        preferred_element_type=jnp.float32)
        q_all = q_all.astype(jnp.bfloat16)

        ii = lax.broadcasted_iota(jnp.int32, (SQ, SKV), 0)
        jj = lax.broadcasted_iota(jnp.int32, (SQ, SKV), 1)
        mask = (jj // 64) <= (ii // 64)

        for b in range(B):
            k2d = k_ref[b].reshape(SKV, CHUNK).astype(jnp.bfloat16)
            v2d = v_ref[b].reshape(SKV, CHUNK).astype(jnp.bfloat16)
            for h in range(HQ_LOCAL):
                qh = q_all[b * SQ:(b + 1) * SQ, h * DH:(h + 1) * DH]
                kh = k2d[:, h * DH:(h + 1) * DH]
                s = lax.dot_general(
                    qh, kh, (((1,), (1,)), ((), ())),
                    preferred_element_type=jnp.float32,
                ) * 0.125
                s = jnp.where(mask, s, -1e9)
                m = jnp.max(s, axis=1, keepdims=True)
                w = jnp.exp(s - m)
                w = (w / jnp.sum(w, axis=1, keepdims=True)).astype(jnp.bfloat16)
                ctx_h = jnp.dot(w, v2d[:, h * DH:(h + 1) * DH],
                                preferred_element_type=jnp.float32)
                comm_ref[0, b * SQ:(b + 1) * SQ, h * DH:(h + 1) * DH] = (
                    ctx_h.astype(jnp.bfloat16))

        rdmas = {}
        for d in range(1, N_DEV):
            rdmas[d] = pltpu.make_async_remote_copy(
                src_ref=comm_ref.at[0],
                dst_ref=comm_ref.at[d],
                send_sem=send_sems.at[d - 1],
                recv_sem=recv_sems.at[d - 1],
                device_id=((me + d) % N_DEV,),
                device_id_type=pl.DeviceIdType.MESH,
            )
            rdmas[d].start()

        wo_mine = wo_ref[pl.ds(me * CHUNK, CHUNK), :].astype(jnp.bfloat16)
        out2d = jnp.dot(comm_ref[0], wo_mine,
                        preferred_element_type=jnp.float32)

        for d in (1, 3, 2):
            rdmas[d].wait_recv()
            origin = (me - d) % N_DEV
            wo_blk = wo_ref[pl.ds(origin * CHUNK, CHUNK), :].astype(
                jnp.bfloat16)
            out2d = out2d + jnp.dot(comm_ref[d], wo_blk,
                                    preferred_element_type=jnp.float32)
        for d in range(1, N_DEV):
            rdmas[d].wait_send()

        out_ref[...] = out2d.reshape(B, SQ, D_MODEL)

    return pl.pallas_call(
        body,
        out_shape=jax.ShapeDtypeStruct((B, SQ, D_MODEL), jnp.float32),
        in_specs=[pl.BlockSpec(memory_space=pltpu.VMEM)] * 5,
        out_specs=pl.BlockSpec(memory_space=pltpu.VMEM),
        scratch_shapes=[
            pltpu.VMEM((N_DEV, ROWS, CHUNK), jnp.bfloat16),
            pltpu.SemaphoreType.DMA((N_DEV - 1,)),
            pltpu.SemaphoreType.DMA((N_DEV - 1,)),
        ],
        compiler_params=pltpu.CompilerParams(collective_id=0),
    )(x, Wq_loc, K_ext, V_ext, Wo)
